# baseline (speedup 1.0000x reference)
import functools

import jax
import jax.numpy as jnp
from jax import lax
from jax.experimental import pallas as pl
from jax.experimental.pallas import tpu as pltpu
from jax.experimental.pallas import tpu_sc as plsc

D = 128
B = 4096
NC = 2
NS = 16
NW = NC * NS
BPW = B // NW

TC_BLOCKS = 8
TB = B // TC_BLOCKS


def _gather_body(table_hbm, uid_hbm, out_hbm, idx_v, rows_v, sem):
    wid = lax.axis_index("s") * NC + lax.axis_index("c")
    base = wid * BPW
    pltpu.sync_copy(uid_hbm.at[pl.ds(base, BPW)], idx_v)
    pltpu.async_copy(table_hbm.at[idx_v], rows_v, sem).wait()
    pltpu.sync_copy(rows_v, out_hbm.at[pl.ds(base, BPW)])


@functools.cache
def _sc_gather():
    return pl.kernel(
        _gather_body,
        out_type=jax.ShapeDtypeStruct((B, D), jnp.float32),
        mesh=plsc.VectorSubcoreMesh(core_axis_name="c", subcore_axis_name="s"),
        scratch_types=[
            pltpu.VMEM((BPW,), jnp.int32),
            pltpu.VMEM((BPW, D), jnp.float32),
            pltpu.SemaphoreType.DMA,
        ],
    )


def _tc_body(user_ref, item_ref, w1_ref, w2_ref, b_ref, out_ref):
    z = (jnp.dot(user_ref[...], w1_ref[...], preferred_element_type=jnp.float32)
         + jnp.dot(item_ref[...], w2_ref[...], preferred_element_type=jnp.float32))
    out_ref[...] = 1.0 / (1.0 + jnp.exp(-(z + b_ref[0, 0])))


def _tc_matvec(user_embs, item_emb, w1, w2, b11):
    return pl.pallas_call(
        _tc_body,
        grid=(TC_BLOCKS,),
        in_specs=[
            pl.BlockSpec((TB, D), lambda i: (i, 0)),
            pl.BlockSpec((TB, D), lambda i: (i, 0)),
            pl.BlockSpec((D, 1), lambda i: (0, 0)),
            pl.BlockSpec((D, 1), lambda i: (0, 0)),
            pl.BlockSpec((1, 1), lambda i: (0, 0), memory_space=pltpu.SMEM),
        ],
        out_specs=pl.BlockSpec((TB, 1), lambda i: (i, 0)),
        out_shape=jax.ShapeDtypeStruct((B, 1), jnp.float32),
        compiler_params=pltpu.CompilerParams(
            dimension_semantics=("arbitrary",)),
    )(user_embs, item_emb, w1, w2, b11)


def kernel(user_id, item_emb, user_table, W, b):
    uid = user_id.astype(jnp.int32)
    user_embs = _sc_gather()(user_table, uid)
    return _tc_matvec(user_embs, item_emb, W[:D], W[D:], b.reshape(1, 1))


# speedup vs baseline: 1.1918x; 1.1918x over previous
import functools

import jax
import jax.numpy as jnp
from jax import lax
from jax.experimental import pallas as pl
from jax.experimental.pallas import tpu as pltpu
from jax.experimental.pallas import tpu_sc as plsc

D = 128
B = 4096
NC = 2
NS = 16
NW = NC * NS
BPW = B // NW

TC_BLOCKS = 8
TB = B // TC_BLOCKS


def _gather_body(table_hbm, uid_hbm, out_hbm, idx_v, rows_v, sem):
    wid = lax.axis_index("s") * NC + lax.axis_index("c")
    base = wid * BPW
    pltpu.sync_copy(uid_hbm.at[pl.ds(base, BPW)], idx_v)
    pltpu.async_copy(table_hbm.at[idx_v], rows_v, sem).wait()
    pltpu.sync_copy(rows_v, out_hbm.at[pl.ds(base, BPW)])


@functools.cache
def _sc_gather():
    return pl.kernel(
        _gather_body,
        out_type=jax.ShapeDtypeStruct((B, D), jnp.float32),
        mesh=plsc.VectorSubcoreMesh(core_axis_name="c", subcore_axis_name="s"),
        scratch_types=[
            pltpu.VMEM((BPW,), jnp.int32),
            pltpu.VMEM((BPW, D), jnp.float32),
            pltpu.SemaphoreType.DMA,
        ],
    )


def _tc_body(user_ref, item_ref, w_ref, b_ref, out_ref):
    s = (user_ref[...] * w_ref[0:1, :] + item_ref[...] * w_ref[1:2, :])
    z = jnp.sum(s, axis=1) + b_ref[0]
    out_ref[...] = 1.0 / (1.0 + jnp.exp(-z))


def _tc_matvec(user_embs, item_emb, w2x128, b1):
    return pl.pallas_call(
        _tc_body,
        in_specs=[
            pl.BlockSpec(memory_space=pltpu.VMEM),
            pl.BlockSpec(memory_space=pltpu.VMEM),
            pl.BlockSpec(memory_space=pltpu.VMEM),
            pl.BlockSpec(memory_space=pltpu.SMEM),
        ],
        out_shape=jax.ShapeDtypeStruct((B,), jnp.float32),
    )(user_embs, item_emb, w2x128, b1)


def kernel(user_id, item_emb, user_table, W, b):
    uid = user_id.astype(jnp.int32)
    user_embs = _sc_gather()(user_table, uid)
    out = _tc_matvec(user_embs, item_emb, W.reshape(2, D), b)
    return out.reshape(B, 1)
